# initial kernel scaffold (unmeasured)
import jax
import jax.numpy as jnp
from jax import lax
from jax.experimental import pallas as pl
from jax.experimental.pallas import tpu as pltpu

N_DEV = 4
MB = 1024
NCOL = 8192
NH = 4096

_CompilerParams = getattr(pltpu, "CompilerParams", None) or getattr(
    pltpu, "TPUCompilerParams"
)


def kernel(x, w_mat):
    x16 = x.astype(jnp.bfloat16)
    w16 = w_mat.astype(jnp.bfloat16)

    def body(
        x_ref,
        w_ref,
        out_ref,
        recv_cw,
        recv_ccw,
        send_cw,
        send_ccw,
        amax_ref,
        sem_send_cw,
        sem_recv_cw,
        sem_send_ccw,
        sem_recv_ccw,
        sem_ax_send,
        sem_ax_recv,
    ):
        p = lax.axis_index("i")
        right = lax.rem(p + 1, N_DEV)
        left = lax.rem(p + N_DEV - 1, N_DEV)

        barrier = pltpu.get_barrier_semaphore()
        pl.semaphore_signal(
            barrier, inc=1, device_id=(left,), device_id_type=pl.DeviceIdType.MESH
        )
        pl.semaphore_signal(
            barrier, inc=1, device_id=(right,), device_id_type=pl.DeviceIdType.MESH
        )
        pl.semaphore_wait(barrier, 2)

        def partial_cw(chunk):
            rows = x_ref[pl.ds(chunk * MB, MB), :]
            return jnp.dot(
                rows, w_ref[:, :NH], preferred_element_type=jnp.float32
            )

        def partial_ccw(chunk):
            rows = x_ref[pl.ds(chunk * MB, MB), :]
            return jnp.dot(
                rows, w_ref[:, NH:], preferred_element_type=jnp.float32
            )

        pc = partial_cw(lax.rem(p + N_DEV - 1, N_DEV))
        qc = partial_ccw(lax.rem(p + 1, N_DEV))

        yl = None
        yr = None
        for s in range(N_DEV - 1):
            if s == 0:
                send_cw[...] = pc.astype(jnp.bfloat16)
                send_ccw[...] = qc.astype(jnp.bfloat16)
            else:
                send_cw[...] = (
                    pc + recv_cw[s - 1].astype(jnp.float32)
                ).astype(jnp.bfloat16)
                send_ccw[...] = (
                    qc + recv_ccw[s - 1].astype(jnp.float32)
                ).astype(jnp.bfloat16)
            r_cw = pltpu.make_async_remote_copy(
                src_ref=send_cw,
                dst_ref=recv_cw.at[s],
                send_sem=sem_send_cw.at[s],
                recv_sem=sem_recv_cw.at[s],
                device_id=(right,),
                device_id_type=pl.DeviceIdType.MESH,
            )
            r_ccw = pltpu.make_async_remote_copy(
                src_ref=send_ccw,
                dst_ref=recv_ccw.at[s],
                send_sem=sem_send_ccw.at[s],
                recv_sem=sem_recv_ccw.at[s],
                device_id=(left,),
                device_id_type=pl.DeviceIdType.MESH,
            )
            r_cw.start()
            r_ccw.start()
            if s < N_DEV - 2:
                pc = partial_cw(lax.rem(p + 2 * N_DEV - 2 - s, N_DEV))
                qc = partial_ccw(lax.rem(p + 2 + s, N_DEV))
            else:
                yl = partial_cw(p)
                yr = partial_ccw(p)
            r_cw.wait()
            r_ccw.wait()

        yl = yl + recv_cw[N_DEV - 2].astype(jnp.float32)
        yr = yr + recv_ccw[N_DEV - 2].astype(jnp.float32)
        out_ref[:, :NH] = yl
        out_ref[:, NH:] = yr

        la = jnp.maximum(jnp.max(jnp.abs(yl)), jnp.max(jnp.abs(yr)))
        amax_ref[0, :, :] = jnp.broadcast_to(la, (8, 128))
        descs = []
        for e in (1, 2, 3):
            tgt = lax.rem(p + N_DEV - e, N_DEV)
            r = pltpu.make_async_remote_copy(
                src_ref=amax_ref.at[0],
                dst_ref=amax_ref.at[e],
                send_sem=sem_ax_send.at[e - 1],
                recv_sem=sem_ax_recv.at[e - 1],
                device_id=(tgt,),
                device_id_type=pl.DeviceIdType.MESH,
            )
            r.start()
            descs.append(r)
        for r in descs:
            r.wait()

        g = jnp.max(amax_ref[...])
        scale = g / 127.0
        inv = 127.0 / g
        out_ref[:, :NH] = (
            jnp.clip(jnp.round(yl * inv), -127.0, 127.0) * scale
        )
        out_ref[:, NH:] = (
            jnp.clip(jnp.round(yr * inv), -127.0, 127.0) * scale
        )

    return pl.pallas_call(
        body,
        out_shape=jax.ShapeDtypeStruct((MB, NCOL), jnp.float32),
        in_specs=[
            pl.BlockSpec(memory_space=pltpu.VMEM),
            pl.BlockSpec(memory_space=pltpu.VMEM),
        ],
        out_specs=pl.BlockSpec(memory_space=pltpu.VMEM),
        scratch_shapes=[
            pltpu.VMEM((N_DEV - 1, MB, NH), jnp.bfloat16),
            pltpu.VMEM((N_DEV - 1, MB, NH), jnp.bfloat16),
            pltpu.VMEM((MB, NH), jnp.bfloat16),
            pltpu.VMEM((MB, NH), jnp.bfloat16),
            pltpu.VMEM((N_DEV, 8, 128), jnp.float32),
            pltpu.SemaphoreType.DMA((N_DEV - 1,)),
            pltpu.SemaphoreType.DMA((N_DEV - 1,)),
            pltpu.SemaphoreType.DMA((N_DEV - 1,)),
            pltpu.SemaphoreType.DMA((N_DEV - 1,)),
            pltpu.SemaphoreType.DMA((N_DEV - 1,)),
            pltpu.SemaphoreType.DMA((N_DEV - 1,)),
        ],
        compiler_params=_CompilerParams(collective_id=0),
    )(x16, w16)


# baseline (device time: 486406 ns/iter reference)
import jax
import jax.numpy as jnp
from jax import lax
from jax.experimental import pallas as pl
from jax.experimental.pallas import tpu as pltpu

N_DEV = 4
MB = 1024
NCOL = 8192
NH = 4096
CT = 1024
NPHASE = NH // CT

_CompilerParams = getattr(pltpu, "CompilerParams", None) or getattr(
    pltpu, "TPUCompilerParams"
)


def kernel(x, w_mat):
    x16 = x.astype(jnp.bfloat16)
    w16 = w_mat.astype(jnp.bfloat16)

    def body(
        x_ref,
        w_ref,
        out_ref,
        recv_cw,
        recv_ccw,
        send_cw,
        send_ccw,
        stage,
        amax_ref,
        sem_send_cw,
        sem_recv_cw,
        sem_send_ccw,
        sem_recv_ccw,
        sem_out,
        sem_ax_send,
        sem_ax_recv,
    ):
        p = lax.axis_index("i")
        right = lax.rem(p + 1, N_DEV)
        left = lax.rem(p + N_DEV - 1, N_DEV)

        barrier = pltpu.get_barrier_semaphore()
        pl.semaphore_signal(
            barrier, inc=1, device_id=(left,), device_id_type=pl.DeviceIdType.MESH
        )
        pl.semaphore_signal(
            barrier, inc=1, device_id=(right,), device_id_type=pl.DeviceIdType.MESH
        )
        pl.semaphore_wait(barrier, 2)

        def partial(chunk, col):
            rows = x_ref[pl.ds(chunk * MB, MB), :]
            return jnp.dot(
                rows, w_ref[:, col : col + CT], preferred_element_type=jnp.float32
            )

        la = jnp.float32(0.0)
        for t in range(NPHASE):
            col_cw = t * CT
            col_ccw = NH + t * CT
            for s in range(N_DEV - 1):
                pc = partial(lax.rem(p + 2 * N_DEV - 1 - s, N_DEV), col_cw)
                if s > 0:
                    pc = pc + recv_cw[s - 1].astype(jnp.float32)
                send_cw[...] = pc.astype(jnp.bfloat16)
                qc = partial(lax.rem(p + 1 + s, N_DEV), col_ccw)
                if s > 0:
                    qc = qc + recv_ccw[s - 1].astype(jnp.float32)
                send_ccw[...] = qc.astype(jnp.bfloat16)
                r_cw = pltpu.make_async_remote_copy(
                    src_ref=send_cw,
                    dst_ref=recv_cw.at[s],
                    send_sem=sem_send_cw.at[s],
                    recv_sem=sem_recv_cw.at[s],
                    device_id=(right,),
                    device_id_type=pl.DeviceIdType.MESH,
                )
                r_ccw = pltpu.make_async_remote_copy(
                    src_ref=send_ccw,
                    dst_ref=recv_ccw.at[s],
                    send_sem=sem_send_ccw.at[s],
                    recv_sem=sem_recv_ccw.at[s],
                    device_id=(left,),
                    device_id_type=pl.DeviceIdType.MESH,
                )
                r_cw.start()
                r_ccw.start()
                r_cw.wait()
                r_ccw.wait()

            yl = partial(p, col_cw) + recv_cw[N_DEV - 2].astype(jnp.float32)
            la = jnp.maximum(la, jnp.max(jnp.abs(yl)))
            stage[...] = yl
            dma_a = pltpu.make_async_copy(
                stage, out_ref.at[:, pl.ds(col_cw, CT)], sem_out.at[0]
            )
            dma_a.start()
            yr = partial(p, col_ccw) + recv_ccw[N_DEV - 2].astype(jnp.float32)
            la = jnp.maximum(la, jnp.max(jnp.abs(yr)))
            dma_a.wait()
            stage[...] = yr
            dma_b = pltpu.make_async_copy(
                stage, out_ref.at[:, pl.ds(col_ccw, CT)], sem_out.at[1]
            )
            dma_b.start()
            dma_b.wait()

        amax_ref[0, :, :] = jnp.broadcast_to(la, (8, 128))
        descs = []
        for e in (1, 2, 3):
            tgt = lax.rem(p + N_DEV - e, N_DEV)
            r = pltpu.make_async_remote_copy(
                src_ref=amax_ref.at[0],
                dst_ref=amax_ref.at[e],
                send_sem=sem_ax_send.at[e - 1],
                recv_sem=sem_ax_recv.at[e - 1],
                device_id=(tgt,),
                device_id_type=pl.DeviceIdType.MESH,
            )
            r.start()
            descs.append(r)
        for r in descs:
            r.wait()

        g = jnp.max(amax_ref[...])
        scale = g / 127.0
        inv = 127.0 / g

        for k in range(NCOL // CT):
            col = k * CT
            st = stage
            sem = sem_out.at[k % 2]
            dma_in = pltpu.make_async_copy(out_ref.at[:, pl.ds(col, CT)], st, sem)
            dma_in.start()
            dma_in.wait()
            st[...] = jnp.clip(jnp.round(st[...] * inv), -127.0, 127.0) * scale
            dma_out = pltpu.make_async_copy(st, out_ref.at[:, pl.ds(col, CT)], sem)
            dma_out.start()
            dma_out.wait()

    return pl.pallas_call(
        body,
        out_shape=jax.ShapeDtypeStruct((MB, NCOL), jnp.float32),
        in_specs=[
            pl.BlockSpec(memory_space=pltpu.VMEM),
            pl.BlockSpec(memory_space=pltpu.VMEM),
        ],
        out_specs=pl.BlockSpec(memory_space=pl.ANY),
        scratch_shapes=[
            pltpu.VMEM((N_DEV - 1, MB, CT), jnp.bfloat16),
            pltpu.VMEM((N_DEV - 1, MB, CT), jnp.bfloat16),
            pltpu.VMEM((MB, CT), jnp.bfloat16),
            pltpu.VMEM((MB, CT), jnp.bfloat16),
            pltpu.VMEM((MB, CT), jnp.float32),
            pltpu.VMEM((N_DEV, 8, 128), jnp.float32),
            pltpu.SemaphoreType.DMA((N_DEV - 1,)),
            pltpu.SemaphoreType.DMA((N_DEV - 1,)),
            pltpu.SemaphoreType.DMA((N_DEV - 1,)),
            pltpu.SemaphoreType.DMA((N_DEV - 1,)),
            pltpu.SemaphoreType.DMA((2,)),
            pltpu.SemaphoreType.DMA((N_DEV - 1,)),
            pltpu.SemaphoreType.DMA((N_DEV - 1,)),
        ],
        compiler_params=_CompilerParams(collective_id=0, vmem_limit_bytes=67_000_000),
    )(x16, w16)


# device time: 420440 ns/iter; 1.1569x vs baseline; 1.1569x over previous
import jax
import jax.numpy as jnp
from jax import lax
from jax.experimental import pallas as pl
from jax.experimental.pallas import tpu as pltpu

N_DEV = 4
MB = 1024
NCOL = 8192
NH = 4096
CT = 1024
NPHASE = NH // CT

_CompilerParams = getattr(pltpu, "CompilerParams", None) or getattr(
    pltpu, "TPUCompilerParams"
)


def kernel(x, w_mat):
    x16 = x.astype(jnp.bfloat16)
    w16 = w_mat.astype(jnp.bfloat16)

    def body(
        x_ref,
        w_ref,
        out_ref,
        recv_cw,
        recv_ccw,
        send_cw,
        send_ccw,
        part_cw,
        part_ccw,
        stage,
        amax_ref,
        sem_send_cw,
        sem_recv_cw,
        sem_send_ccw,
        sem_recv_ccw,
        sem_out,
        sem_ax_send,
        sem_ax_recv,
    ):
        p = lax.axis_index("i")
        right = lax.rem(p + 1, N_DEV)
        left = lax.rem(p + N_DEV - 1, N_DEV)

        barrier = pltpu.get_barrier_semaphore()
        pl.semaphore_signal(
            barrier, inc=1, device_id=(left,), device_id_type=pl.DeviceIdType.MESH
        )
        pl.semaphore_signal(
            barrier, inc=1, device_id=(right,), device_id_type=pl.DeviceIdType.MESH
        )
        pl.semaphore_wait(barrier, 2)

        def partial(chunk, col):
            rows = x_ref[pl.ds(chunk * MB, MB), :]
            return jnp.dot(
                rows, w_ref[:, col : col + CT], preferred_element_type=jnp.float32
            )

        def finals(t, la):
            col_cw = t * CT
            col_ccw = NH + t * CT
            yl = partial(p, col_cw) + recv_cw[N_DEV - 2].astype(jnp.float32)
            la = jnp.maximum(la, jnp.max(jnp.abs(yl)))
            stage[...] = yl
            dma_a = pltpu.make_async_copy(
                stage, out_ref.at[:, pl.ds(col_cw, CT)], sem_out.at[0]
            )
            dma_a.start()
            yr = partial(p, col_ccw) + recv_ccw[N_DEV - 2].astype(jnp.float32)
            la = jnp.maximum(la, jnp.max(jnp.abs(yr)))
            dma_a.wait()
            stage[...] = yr
            dma_b = pltpu.make_async_copy(
                stage, out_ref.at[:, pl.ds(col_ccw, CT)], sem_out.at[1]
            )
            dma_b.start()
            dma_b.wait()
            return la

        la = jnp.float32(0.0)
        part_cw[...] = partial(lax.rem(p + N_DEV - 1, N_DEV), 0).astype(
            jnp.bfloat16
        )
        part_ccw[...] = partial(lax.rem(p + 1, N_DEV), NH).astype(jnp.bfloat16)
        for g in range(NPHASE * (N_DEV - 1)):
            t, s = divmod(g, N_DEV - 1)
            col_cw = t * CT
            col_ccw = NH + t * CT
            if s == 0:
                send_cw[...] = part_cw[...]
                send_ccw[...] = part_ccw[...]
            else:
                send_cw[...] = (
                    part_cw[...].astype(jnp.float32)
                    + recv_cw[s - 1].astype(jnp.float32)
                ).astype(jnp.bfloat16)
                send_ccw[...] = (
                    part_ccw[...].astype(jnp.float32)
                    + recv_ccw[s - 1].astype(jnp.float32)
                ).astype(jnp.bfloat16)
            r_cw = pltpu.make_async_remote_copy(
                src_ref=send_cw,
                dst_ref=recv_cw.at[s],
                send_sem=sem_send_cw.at[s],
                recv_sem=sem_recv_cw.at[s],
                device_id=(right,),
                device_id_type=pl.DeviceIdType.MESH,
            )
            r_ccw = pltpu.make_async_remote_copy(
                src_ref=send_ccw,
                dst_ref=recv_ccw.at[s],
                send_sem=sem_send_ccw.at[s],
                recv_sem=sem_recv_ccw.at[s],
                device_id=(left,),
                device_id_type=pl.DeviceIdType.MESH,
            )
            r_cw.start()
            r_ccw.start()
            if s < N_DEV - 2:
                part_cw[...] = partial(
                    lax.rem(p + 2 * N_DEV - 2 - s, N_DEV), col_cw
                ).astype(jnp.bfloat16)
                part_ccw[...] = partial(
                    lax.rem(p + 2 + s, N_DEV), col_ccw
                ).astype(jnp.bfloat16)
            elif t < NPHASE - 1:
                part_cw[...] = partial(
                    lax.rem(p + N_DEV - 1, N_DEV), col_cw + CT
                ).astype(jnp.bfloat16)
                part_ccw[...] = partial(
                    lax.rem(p + 1, N_DEV), col_ccw + CT
                ).astype(jnp.bfloat16)
            if s == 0 and t > 0:
                la = finals(t - 1, la)
            r_cw.wait()
            r_ccw.wait()
        la = finals(NPHASE - 1, la)

        amax_ref[0, :, :] = jnp.broadcast_to(la, (8, 128))
        descs = []
        for e in (1, 2, 3):
            tgt = lax.rem(p + N_DEV - e, N_DEV)
            r = pltpu.make_async_remote_copy(
                src_ref=amax_ref.at[0],
                dst_ref=amax_ref.at[e],
                send_sem=sem_ax_send.at[e - 1],
                recv_sem=sem_ax_recv.at[e - 1],
                device_id=(tgt,),
                device_id_type=pl.DeviceIdType.MESH,
            )
            r.start()
            descs.append(r)
        for r in descs:
            r.wait()

        g = jnp.max(amax_ref[...])
        scale = g / 127.0
        inv = 127.0 / g

        for k in range(NCOL // CT):
            col = k * CT
            st = stage
            sem = sem_out.at[k % 2]
            dma_in = pltpu.make_async_copy(out_ref.at[:, pl.ds(col, CT)], st, sem)
            dma_in.start()
            dma_in.wait()
            st[...] = jnp.clip(jnp.round(st[...] * inv), -127.0, 127.0) * scale
            dma_out = pltpu.make_async_copy(st, out_ref.at[:, pl.ds(col, CT)], sem)
            dma_out.start()
            dma_out.wait()

    return pl.pallas_call(
        body,
        out_shape=jax.ShapeDtypeStruct((MB, NCOL), jnp.float32),
        in_specs=[
            pl.BlockSpec(memory_space=pltpu.VMEM),
            pl.BlockSpec(memory_space=pltpu.VMEM),
        ],
        out_specs=pl.BlockSpec(memory_space=pl.ANY),
        scratch_shapes=[
            pltpu.VMEM((N_DEV - 1, MB, CT), jnp.bfloat16),
            pltpu.VMEM((N_DEV - 1, MB, CT), jnp.bfloat16),
            pltpu.VMEM((MB, CT), jnp.bfloat16),
            pltpu.VMEM((MB, CT), jnp.bfloat16),
            pltpu.VMEM((MB, CT), jnp.bfloat16),
            pltpu.VMEM((MB, CT), jnp.bfloat16),
            pltpu.VMEM((MB, CT), jnp.float32),
            pltpu.VMEM((N_DEV, 8, 128), jnp.float32),
            pltpu.SemaphoreType.DMA((N_DEV - 1,)),
            pltpu.SemaphoreType.DMA((N_DEV - 1,)),
            pltpu.SemaphoreType.DMA((N_DEV - 1,)),
            pltpu.SemaphoreType.DMA((N_DEV - 1,)),
            pltpu.SemaphoreType.DMA((2,)),
            pltpu.SemaphoreType.DMA((N_DEV - 1,)),
            pltpu.SemaphoreType.DMA((N_DEV - 1,)),
        ],
        compiler_params=_CompilerParams(collective_id=0, vmem_limit_bytes=67_000_000),
    )(x16, w16)


# device time: 384305 ns/iter; 1.2657x vs baseline; 1.0940x over previous
import jax
import jax.numpy as jnp
from jax import lax
from jax.experimental import pallas as pl
from jax.experimental.pallas import tpu as pltpu

N_DEV = 4
MB = 1024
NCOL = 8192
NH = 4096
CT = 1024
SUB = 512
NPHASE = NH // CT

_CompilerParams = getattr(pltpu, "CompilerParams", None) or getattr(
    pltpu, "TPUCompilerParams"
)


def kernel(x, w_mat):
    x16 = x.astype(jnp.bfloat16)
    w16 = w_mat.astype(jnp.bfloat16)

    def body(
        x_ref,
        w_ref,
        out_ref,
        recv_cw,
        recv_ccw,
        send_cw,
        send_ccw,
        part_cw,
        part_ccw,
        stg0,
        stg1,
        amax_ref,
        sem_send_cw,
        sem_recv_cw,
        sem_send_ccw,
        sem_recv_ccw,
        sem_out,
        sem_ax_send,
        sem_ax_recv,
    ):
        p = lax.axis_index("i")
        right = lax.rem(p + 1, N_DEV)
        left = lax.rem(p + N_DEV - 1, N_DEV)

        barrier = pltpu.get_barrier_semaphore()
        pl.semaphore_signal(
            barrier, inc=1, device_id=(left,), device_id_type=pl.DeviceIdType.MESH
        )
        pl.semaphore_signal(
            barrier, inc=1, device_id=(right,), device_id_type=pl.DeviceIdType.MESH
        )
        pl.semaphore_wait(barrier, 2)

        def partial(chunk, col, width=CT):
            rows = x_ref[pl.ds(chunk * MB, MB), :]
            return jnp.dot(
                rows,
                w_ref[:, col : col + width],
                preferred_element_type=jnp.float32,
            )

        def write_send(dirn, s, j):
            lo = j * SUB
            send = send_cw if dirn == 0 else send_ccw
            part = part_cw if dirn == 0 else part_ccw
            recv = recv_cw if dirn == 0 else recv_ccw
            if s == 0:
                send[:, lo : lo + SUB] = part[:, lo : lo + SUB]
            else:
                send[:, lo : lo + SUB] = (
                    part[:, lo : lo + SUB].astype(jnp.float32)
                    + recv[s - 1, :, lo : lo + SUB].astype(jnp.float32)
                ).astype(jnp.bfloat16)

        def start_sub(dirn, s, j):
            lo = j * SUB
            send = send_cw if dirn == 0 else send_ccw
            recv = recv_cw if dirn == 0 else recv_ccw
            ssem = sem_send_cw if dirn == 0 else sem_send_ccw
            rsem = sem_recv_cw if dirn == 0 else sem_recv_ccw
            tgt = right if dirn == 0 else left
            r = pltpu.make_async_remote_copy(
                src_ref=send.at[:, pl.ds(lo, SUB)],
                dst_ref=recv.at[s, :, pl.ds(lo, SUB)],
                send_sem=ssem.at[s, j],
                recv_sem=rsem.at[s, j],
                device_id=(tgt,),
                device_id_type=pl.DeviceIdType.MESH,
            )
            r.start()
            return r

        def finals(t, la):
            col_cw = t * CT
            col_ccw = NH + t * CT
            yl = partial(p, col_cw) + recv_cw[N_DEV - 2].astype(jnp.float32)
            la = jnp.maximum(la, jnp.max(jnp.abs(yl)))
            stg0[...] = yl[:, :SUB]
            d0 = pltpu.make_async_copy(
                stg0, out_ref.at[:, pl.ds(col_cw, SUB)], sem_out.at[0]
            )
            d0.start()
            stg1[...] = yl[:, SUB:]
            d1 = pltpu.make_async_copy(
                stg1, out_ref.at[:, pl.ds(col_cw + SUB, SUB)], sem_out.at[1]
            )
            d1.start()
            yr = partial(p, col_ccw) + recv_ccw[N_DEV - 2].astype(jnp.float32)
            la = jnp.maximum(la, jnp.max(jnp.abs(yr)))
            d0.wait()
            stg0[...] = yr[:, :SUB]
            d2 = pltpu.make_async_copy(
                stg0, out_ref.at[:, pl.ds(col_ccw, SUB)], sem_out.at[0]
            )
            d2.start()
            d1.wait()
            stg1[...] = yr[:, SUB:]
            d3 = pltpu.make_async_copy(
                stg1, out_ref.at[:, pl.ds(col_ccw + SUB, SUB)], sem_out.at[1]
            )
            d3.start()
            d2.wait()
            d3.wait()
            return la

        la = jnp.float32(0.0)
        part_cw[...] = partial(lax.rem(p + N_DEV - 1, N_DEV), 0).astype(
            jnp.bfloat16
        )
        part_ccw[...] = partial(lax.rem(p + 1, N_DEV), NH).astype(jnp.bfloat16)
        prev = None
        for g in range(NPHASE * (N_DEV - 1)):
            t, s = divmod(g, N_DEV - 1)
            col_cw = t * CT
            col_ccw = NH + t * CT
            if prev is not None:
                prev[0].wait()
                prev[1].wait()
            write_send(0, s, 0)
            d_lcw = start_sub(0, s, 0)
            write_send(1, s, 0)
            d_lccw = start_sub(1, s, 0)
            if prev is not None:
                prev[2].wait()
                prev[3].wait()
            write_send(0, s, 1)
            d_rcw = start_sub(0, s, 1)
            write_send(1, s, 1)
            d_rccw = start_sub(1, s, 1)
            prev = (d_lcw, d_lccw, d_rcw, d_rccw)
            if s < N_DEV - 2:
                part_cw[...] = partial(
                    lax.rem(p + 2 * N_DEV - 2 - s, N_DEV), col_cw
                ).astype(jnp.bfloat16)
                part_ccw[...] = partial(
                    lax.rem(p + 2 + s, N_DEV), col_ccw
                ).astype(jnp.bfloat16)
            elif t < NPHASE - 1:
                part_cw[...] = partial(
                    lax.rem(p + N_DEV - 1, N_DEV), col_cw + CT
                ).astype(jnp.bfloat16)
                part_ccw[...] = partial(
                    lax.rem(p + 1, N_DEV), col_ccw + CT
                ).astype(jnp.bfloat16)
            if s == 0 and t > 0:
                la = finals(t - 1, la)
        for d in prev:
            d.wait()
        la = finals(NPHASE - 1, la)

        amax_ref[0, :, :] = jnp.broadcast_to(la, (8, 128))
        descs = []
        for e in (1, 2, 3):
            tgt = lax.rem(p + N_DEV - e, N_DEV)
            r = pltpu.make_async_remote_copy(
                src_ref=amax_ref.at[0],
                dst_ref=amax_ref.at[e],
                send_sem=sem_ax_send.at[e - 1],
                recv_sem=sem_ax_recv.at[e - 1],
                device_id=(tgt,),
                device_id_type=pl.DeviceIdType.MESH,
            )
            r.start()
            descs.append(r)
        for r in descs:
            r.wait()

        gmax = jnp.max(amax_ref[...])
        scale = gmax / 127.0
        inv = 127.0 / gmax

        stages = (stg0, stg1)
        n_tiles = NCOL // SUB
        d_in = pltpu.make_async_copy(
            out_ref.at[:, pl.ds(0, SUB)], stg0, sem_out.at[0]
        )
        d_in.start()
        d_outs = [None, None]
        for k in range(n_tiles):
            d_in.wait()
            cur = stages[k % 2]
            if k + 1 < n_tiles:
                if d_outs[(k + 1) % 2] is not None:
                    d_outs[(k + 1) % 2].wait()
                d_in = pltpu.make_async_copy(
                    out_ref.at[:, pl.ds((k + 1) * SUB, SUB)],
                    stages[(k + 1) % 2],
                    sem_out.at[(k + 1) % 2],
                )
                d_in.start()
            cur[...] = jnp.clip(jnp.round(cur[...] * inv), -127.0, 127.0) * scale
            d_out = pltpu.make_async_copy(
                cur, out_ref.at[:, pl.ds(k * SUB, SUB)], sem_out.at[2 + k % 2]
            )
            d_out.start()
            d_outs[k % 2] = d_out
        for d in d_outs:
            if d is not None:
                d.wait()

    return pl.pallas_call(
        body,
        out_shape=jax.ShapeDtypeStruct((MB, NCOL), jnp.float32),
        in_specs=[
            pl.BlockSpec(memory_space=pltpu.VMEM),
            pl.BlockSpec(memory_space=pltpu.VMEM),
        ],
        out_specs=pl.BlockSpec(memory_space=pl.ANY),
        scratch_shapes=[
            pltpu.VMEM((N_DEV - 1, MB, CT), jnp.bfloat16),
            pltpu.VMEM((N_DEV - 1, MB, CT), jnp.bfloat16),
            pltpu.VMEM((MB, CT), jnp.bfloat16),
            pltpu.VMEM((MB, CT), jnp.bfloat16),
            pltpu.VMEM((MB, CT), jnp.bfloat16),
            pltpu.VMEM((MB, CT), jnp.bfloat16),
            pltpu.VMEM((MB, SUB), jnp.float32),
            pltpu.VMEM((MB, SUB), jnp.float32),
            pltpu.VMEM((N_DEV, 8, 128), jnp.float32),
            pltpu.SemaphoreType.DMA((N_DEV - 1, 2)),
            pltpu.SemaphoreType.DMA((N_DEV - 1, 2)),
            pltpu.SemaphoreType.DMA((N_DEV - 1, 2)),
            pltpu.SemaphoreType.DMA((N_DEV - 1, 2)),
            pltpu.SemaphoreType.DMA((4,)),
            pltpu.SemaphoreType.DMA((N_DEV - 1,)),
            pltpu.SemaphoreType.DMA((N_DEV - 1,)),
        ],
        compiler_params=_CompilerParams(collective_id=0, vmem_limit_bytes=67_000_000),
    )(x16, w16)


# device time: 379429 ns/iter; 1.2819x vs baseline; 1.0129x over previous
import jax
import jax.numpy as jnp
from jax import lax
from jax.experimental import pallas as pl
from jax.experimental.pallas import tpu as pltpu

N_DEV = 4
MB = 1024
NCOL = 8192
NH = 4096
CT = 1024
SUB = 512
NPHASE = NH // CT

_CompilerParams = getattr(pltpu, "CompilerParams", None) or getattr(
    pltpu, "TPUCompilerParams"
)


def kernel(x, w_mat):
    x16 = x.astype(jnp.bfloat16)
    w16 = w_mat.astype(jnp.bfloat16)

    def body(
        x_ref,
        w_ref,
        out_ref,
        recv_cw,
        recv_ccw,
        send_cw,
        send_ccw,
        part_cw,
        part_ccw,
        fin_cw,
        fin_ccw,
        stg0,
        stg1,
        amax_ref,
        sem_send_cw,
        sem_recv_cw,
        sem_send_ccw,
        sem_recv_ccw,
        sem_out,
        sem_ax_send,
        sem_ax_recv,
    ):
        p = lax.axis_index("i")
        right = lax.rem(p + 1, N_DEV)
        left = lax.rem(p + N_DEV - 1, N_DEV)

        barrier = pltpu.get_barrier_semaphore()
        pl.semaphore_signal(
            barrier, inc=1, device_id=(left,), device_id_type=pl.DeviceIdType.MESH
        )
        pl.semaphore_signal(
            barrier, inc=1, device_id=(right,), device_id_type=pl.DeviceIdType.MESH
        )
        pl.semaphore_wait(barrier, 2)

        def partial(chunk, col, width=CT):
            rows = x_ref[pl.ds(chunk * MB, MB), :]
            return jnp.dot(
                rows,
                w_ref[:, col : col + width],
                preferred_element_type=jnp.float32,
            )

        def write_send(dirn, s, j):
            lo = j * SUB
            send = send_cw if dirn == 0 else send_ccw
            part = part_cw if dirn == 0 else part_ccw
            recv = recv_cw if dirn == 0 else recv_ccw
            if s == 0:
                send[:, lo : lo + SUB] = part[:, lo : lo + SUB]
            else:
                send[:, lo : lo + SUB] = (
                    part[:, lo : lo + SUB].astype(jnp.float32)
                    + recv[s - 1, :, lo : lo + SUB].astype(jnp.float32)
                ).astype(jnp.bfloat16)

        def start_sub(dirn, s, j):
            lo = j * SUB
            send = send_cw if dirn == 0 else send_ccw
            recv = recv_cw if dirn == 0 else recv_ccw
            ssem = sem_send_cw if dirn == 0 else sem_send_ccw
            rsem = sem_recv_cw if dirn == 0 else sem_recv_ccw
            tgt = right if dirn == 0 else left
            r = pltpu.make_async_remote_copy(
                src_ref=send.at[:, pl.ds(lo, SUB)],
                dst_ref=recv.at[s, :, pl.ds(lo, SUB)],
                send_sem=ssem.at[s, j],
                recv_sem=rsem.at[s, j],
                device_id=(tgt,),
                device_id_type=pl.DeviceIdType.MESH,
            )
            r.start()
            return r

        def finals(t, la):
            col_cw = t * CT
            col_ccw = NH + t * CT
            yl = fin_cw[...].astype(jnp.float32) + recv_cw[N_DEV - 2].astype(
                jnp.float32
            )
            la = jnp.maximum(la, jnp.max(jnp.abs(yl)))
            stg0[...] = yl[:, :SUB]
            d0 = pltpu.make_async_copy(
                stg0, out_ref.at[:, pl.ds(col_cw, SUB)], sem_out.at[0]
            )
            d0.start()
            stg1[...] = yl[:, SUB:]
            d1 = pltpu.make_async_copy(
                stg1, out_ref.at[:, pl.ds(col_cw + SUB, SUB)], sem_out.at[1]
            )
            d1.start()
            yr = fin_ccw[...].astype(jnp.float32) + recv_ccw[N_DEV - 2].astype(
                jnp.float32
            )
            la = jnp.maximum(la, jnp.max(jnp.abs(yr)))
            d0.wait()
            stg0[...] = yr[:, :SUB]
            d2 = pltpu.make_async_copy(
                stg0, out_ref.at[:, pl.ds(col_ccw, SUB)], sem_out.at[0]
            )
            d2.start()
            d1.wait()
            stg1[...] = yr[:, SUB:]
            d3 = pltpu.make_async_copy(
                stg1, out_ref.at[:, pl.ds(col_ccw + SUB, SUB)], sem_out.at[1]
            )
            d3.start()
            d2.wait()
            d3.wait()
            return la

        la = jnp.float32(0.0)
        c0_cw = lax.rem(p + N_DEV - 1, N_DEV)
        c0_ccw = lax.rem(p + 1, N_DEV)
        prev = None
        for g in range(NPHASE * (N_DEV - 1)):
            t, s = divmod(g, N_DEV - 1)
            col_cw = t * CT
            col_ccw = NH + t * CT
            if g == 0:
                send_cw[:, :SUB] = partial(c0_cw, 0, SUB).astype(jnp.bfloat16)
                d_lcw = start_sub(0, 0, 0)
                send_ccw[:, :SUB] = partial(c0_ccw, NH, SUB).astype(
                    jnp.bfloat16
                )
                d_lccw = start_sub(1, 0, 0)
                send_cw[:, SUB:] = partial(c0_cw, SUB, SUB).astype(
                    jnp.bfloat16
                )
                d_rcw = start_sub(0, 0, 1)
                send_ccw[:, SUB:] = partial(c0_ccw, NH + SUB, SUB).astype(
                    jnp.bfloat16
                )
                d_rccw = start_sub(1, 0, 1)
            else:
                prev[0].wait()
                prev[1].wait()
                write_send(0, s, 0)
                d_lcw = start_sub(0, s, 0)
                write_send(1, s, 0)
                d_lccw = start_sub(1, s, 0)
                prev[2].wait()
                prev[3].wait()
                write_send(0, s, 1)
                d_rcw = start_sub(0, s, 1)
                write_send(1, s, 1)
                d_rccw = start_sub(1, s, 1)
            prev = (d_lcw, d_lccw, d_rcw, d_rccw)
            if s < N_DEV - 2:
                part_cw[...] = partial(
                    lax.rem(p + 2 * N_DEV - 2 - s, N_DEV), col_cw
                ).astype(jnp.bfloat16)
                part_ccw[...] = partial(
                    lax.rem(p + 2 + s, N_DEV), col_ccw
                ).astype(jnp.bfloat16)
            elif t < NPHASE - 1:
                part_cw[...] = partial(
                    lax.rem(p + N_DEV - 1, N_DEV), col_cw + CT
                ).astype(jnp.bfloat16)
                part_ccw[...] = partial(
                    lax.rem(p + 1, N_DEV), col_ccw + CT
                ).astype(jnp.bfloat16)
            if s == 1:
                fin_cw[...] = partial(p, col_cw).astype(jnp.bfloat16)
            elif s == 2:
                fin_ccw[...] = partial(p, col_ccw).astype(jnp.bfloat16)
            if s == 0 and t > 0:
                la = finals(t - 1, la)
        for d in prev:
            d.wait()
        la = finals(NPHASE - 1, la)

        amax_ref[0, :, :] = jnp.broadcast_to(la, (8, 128))
        descs = []
        for e in (1, 2, 3):
            tgt = lax.rem(p + N_DEV - e, N_DEV)
            r = pltpu.make_async_remote_copy(
                src_ref=amax_ref.at[0],
                dst_ref=amax_ref.at[e],
                send_sem=sem_ax_send.at[e - 1],
                recv_sem=sem_ax_recv.at[e - 1],
                device_id=(tgt,),
                device_id_type=pl.DeviceIdType.MESH,
            )
            r.start()
            descs.append(r)
        for r in descs:
            r.wait()

        gmax = jnp.max(amax_ref[...])
        scale = gmax / 127.0
        inv = 127.0 / gmax

        stages = (stg0, stg1)
        n_tiles = NCOL // SUB
        d_in = pltpu.make_async_copy(
            out_ref.at[:, pl.ds(0, SUB)], stg0, sem_out.at[0]
        )
        d_in.start()
        d_outs = [None, None]
        for k in range(n_tiles):
            d_in.wait()
            cur = stages[k % 2]
            if k + 1 < n_tiles:
                if d_outs[(k + 1) % 2] is not None:
                    d_outs[(k + 1) % 2].wait()
                d_in = pltpu.make_async_copy(
                    out_ref.at[:, pl.ds((k + 1) * SUB, SUB)],
                    stages[(k + 1) % 2],
                    sem_out.at[(k + 1) % 2],
                )
                d_in.start()
            cur[...] = jnp.clip(jnp.round(cur[...] * inv), -127.0, 127.0) * scale
            d_out = pltpu.make_async_copy(
                cur, out_ref.at[:, pl.ds(k * SUB, SUB)], sem_out.at[2 + k % 2]
            )
            d_out.start()
            d_outs[k % 2] = d_out
        for d in d_outs:
            if d is not None:
                d.wait()

    return pl.pallas_call(
        body,
        out_shape=jax.ShapeDtypeStruct((MB, NCOL), jnp.float32),
        in_specs=[
            pl.BlockSpec(memory_space=pltpu.VMEM),
            pl.BlockSpec(memory_space=pltpu.VMEM),
        ],
        out_specs=pl.BlockSpec(memory_space=pl.ANY),
        scratch_shapes=[
            pltpu.VMEM((N_DEV - 1, MB, CT), jnp.bfloat16),
            pltpu.VMEM((N_DEV - 1, MB, CT), jnp.bfloat16),
            pltpu.VMEM((MB, CT), jnp.bfloat16),
            pltpu.VMEM((MB, CT), jnp.bfloat16),
            pltpu.VMEM((MB, CT), jnp.bfloat16),
            pltpu.VMEM((MB, CT), jnp.bfloat16),
            pltpu.VMEM((MB, CT), jnp.bfloat16),
            pltpu.VMEM((MB, CT), jnp.bfloat16),
            pltpu.VMEM((MB, SUB), jnp.float32),
            pltpu.VMEM((MB, SUB), jnp.float32),
            pltpu.VMEM((N_DEV, 8, 128), jnp.float32),
            pltpu.SemaphoreType.DMA((N_DEV - 1, 2)),
            pltpu.SemaphoreType.DMA((N_DEV - 1, 2)),
            pltpu.SemaphoreType.DMA((N_DEV - 1, 2)),
            pltpu.SemaphoreType.DMA((N_DEV - 1, 2)),
            pltpu.SemaphoreType.DMA((4,)),
            pltpu.SemaphoreType.DMA((N_DEV - 1,)),
            pltpu.SemaphoreType.DMA((N_DEV - 1,)),
        ],
        compiler_params=_CompilerParams(collective_id=0, vmem_limit_bytes=67_000_000),
    )(x16, w16)
